# initial kernel scaffold (unmeasured)
import jax
import jax.numpy as jnp
from jax import lax
from jax.experimental import pallas as pl
from jax.experimental.pallas import tpu as pltpu

N_GLOBAL = 2048
EPS = 1e-5


def kernel(x, gamma, beta):
    m, n = x.shape
    gamma2 = gamma.reshape(1, n)
    beta2 = beta.reshape(1, n)

    def body(x_ref, g_ref, b_ref, o_ref, send_buf, recv_buf, send_sem, recv_sem):
        my_x = lax.axis_index("x")
        my_y = lax.axis_index("y")
        peer = (my_x, 1 - my_y)

        barrier = pltpu.get_barrier_semaphore()
        pl.semaphore_signal(
            barrier, inc=1, device_id=peer, device_id_type=pl.DeviceIdType.MESH
        )
        pl.semaphore_wait(barrier, 1)

        xv = x_ref[:, :]
        s1 = jnp.sum(xv, axis=1, keepdims=True)
        s2 = jnp.sum(xv * xv, axis=1, keepdims=True)
        send_buf[:, 0:1] = s1
        send_buf[:, 1:2] = s2

        rdma = pltpu.make_async_remote_copy(
            src_ref=send_buf,
            dst_ref=recv_buf,
            send_sem=send_sem,
            recv_sem=recv_sem,
            device_id=peer,
            device_id_type=pl.DeviceIdType.MESH,
        )
        rdma.start()
        rdma.wait()

        tot1 = s1 + recv_buf[:, 0:1]
        tot2 = s2 + recv_buf[:, 1:2]
        mean = tot1 / N_GLOBAL
        var = tot2 / N_GLOBAL - mean * mean
        inv = lax.rsqrt(var + EPS)
        o_ref[:, :] = g_ref[:, :] * ((xv - mean) * inv) + b_ref[:, :]

    return pl.pallas_call(
        body,
        out_shape=jax.ShapeDtypeStruct((m, n), jnp.float32),
        in_specs=[pl.BlockSpec(memory_space=pltpu.VMEM)] * 3,
        out_specs=pl.BlockSpec(memory_space=pltpu.VMEM),
        scratch_shapes=[
            pltpu.VMEM((m, 2), jnp.float32),
            pltpu.VMEM((m, 2), jnp.float32),
            pltpu.SemaphoreType.DMA,
            pltpu.SemaphoreType.DMA,
        ],
        compiler_params=pltpu.CompilerParams(collective_id=0),
    )(x, gamma2, beta2)


# baseline (device time: 56594 ns/iter reference)
import jax
import jax.numpy as jnp
from jax import lax
from jax.experimental import pallas as pl
from jax.experimental.pallas import tpu as pltpu

N_GLOBAL = 2048
EPS = 1e-5


def kernel(x, gamma, beta):
    m, n = x.shape
    gamma2 = gamma.reshape(1, n)
    beta2 = beta.reshape(1, n)

    def body(x_ref, g_ref, b_ref, o_ref, send_buf, recv_buf, send_sem, recv_sem):
        my_x = lax.axis_index("x")
        my_y = lax.axis_index("y")
        peer = (my_x, 1 - my_y)

        barrier = pltpu.get_barrier_semaphore()
        pl.semaphore_signal(
            barrier, inc=1, device_id=peer, device_id_type=pl.DeviceIdType.MESH
        )
        pl.semaphore_wait(barrier, 1)

        xv = x_ref[:, :]
        s1 = jnp.sum(xv, axis=1, keepdims=True)
        s2 = jnp.sum(xv * xv, axis=1, keepdims=True)
        send_buf[:, 0:1] = s1
        send_buf[:, 1:2] = s2

        rdma = pltpu.make_async_remote_copy(
            src_ref=send_buf,
            dst_ref=recv_buf,
            send_sem=send_sem,
            recv_sem=recv_sem,
            device_id=peer,
            device_id_type=pl.DeviceIdType.MESH,
        )
        rdma.start()
        rdma.wait()

        tot1 = s1 + recv_buf[:, 0:1]
        tot2 = s2 + recv_buf[:, 1:2]
        mean = tot1 / N_GLOBAL
        var = tot2 / N_GLOBAL - mean * mean
        inv = lax.rsqrt(var + EPS)
        o_ref[:, :] = g_ref[:, :] * ((xv - mean) * inv) + b_ref[:, :]

    return pl.pallas_call(
        body,
        out_shape=jax.ShapeDtypeStruct((m, n), jnp.float32),
        in_specs=[pl.BlockSpec(memory_space=pltpu.VMEM)] * 3,
        out_specs=pl.BlockSpec(memory_space=pltpu.VMEM),
        scratch_shapes=[
            pltpu.VMEM((m, 2), jnp.float32),
            pltpu.VMEM((m, 2), jnp.float32),
            pltpu.SemaphoreType.DMA,
            pltpu.SemaphoreType.DMA,
        ],
        compiler_params=pltpu.CompilerParams(
            collective_id=0, vmem_limit_bytes=64 * 1024 * 1024
        ),
    )(x, gamma2, beta2)


# device time: 41563 ns/iter; 1.3616x vs baseline; 1.3616x over previous
import jax
import jax.numpy as jnp
from jax import lax
from jax.experimental import pallas as pl
from jax.experimental.pallas import tpu as pltpu

N_GLOBAL = 2048
EPS = 1e-5
K = 8


def kernel(x, gamma, beta):
    m, n = x.shape
    bm = m // K
    gamma2 = gamma.reshape(1, n)
    beta2 = beta.reshape(1, n)

    def stats_body(x_ref, stats_ref, part, recv, send_sem, recv_sem):
        k = pl.program_id(0)
        xv = x_ref[...]
        s1 = jnp.sum(xv, axis=1, keepdims=True)
        s2 = jnp.sum(xv * xv, axis=1, keepdims=True)
        part[pl.ds(k * bm, bm), 0:1] = s1
        part[pl.ds(k * bm, bm), 1:2] = s2

        @pl.when(k == K - 1)
        def _():
            my_x = lax.axis_index("x")
            my_y = lax.axis_index("y")
            peer = (my_x, 1 - my_y)
            barrier = pltpu.get_barrier_semaphore()
            pl.semaphore_signal(
                barrier, inc=1, device_id=peer,
                device_id_type=pl.DeviceIdType.MESH,
            )
            pl.semaphore_wait(barrier, 1)

            rdma = pltpu.make_async_remote_copy(
                src_ref=part,
                dst_ref=recv,
                send_sem=send_sem,
                recv_sem=recv_sem,
                device_id=peer,
                device_id_type=pl.DeviceIdType.MESH,
            )
            rdma.start()
            rdma.wait()

            tot1 = part[:, 0:1] + recv[:, 0:1]
            tot2 = part[:, 1:2] + recv[:, 1:2]
            mean = tot1 / N_GLOBAL
            var = tot2 / N_GLOBAL - mean * mean
            stats_ref[:, 0:1] = mean
            stats_ref[:, 1:2] = lax.rsqrt(var + EPS)

    stats = pl.pallas_call(
        stats_body,
        grid=(K,),
        out_shape=jax.ShapeDtypeStruct((m, 2), jnp.float32),
        in_specs=[pl.BlockSpec((bm, n), lambda k: (k, 0))],
        out_specs=pl.BlockSpec((m, 2), lambda k: (0, 0)),
        scratch_shapes=[
            pltpu.VMEM((m, 2), jnp.float32),
            pltpu.VMEM((m, 2), jnp.float32),
            pltpu.SemaphoreType.DMA,
            pltpu.SemaphoreType.DMA,
        ],
        compiler_params=pltpu.CompilerParams(collective_id=0),
    )(x)

    def norm_body(x_ref, stats_ref, g_ref, b_ref, o_ref):
        k = pl.program_id(0)
        mean = stats_ref[pl.ds(k * bm, bm), 0:1]
        rstd = stats_ref[pl.ds(k * bm, bm), 1:2]
        o_ref[...] = (
            (x_ref[...] - mean) * rstd * g_ref[...] + b_ref[...]
        ).astype(jnp.bfloat16)

    return pl.pallas_call(
        norm_body,
        grid=(K,),
        out_shape=jax.ShapeDtypeStruct((m, n), jnp.bfloat16),
        in_specs=[
            pl.BlockSpec((bm, n), lambda k: (k, 0)),
            pl.BlockSpec((m, 2), lambda k: (0, 0)),
            pl.BlockSpec((1, n), lambda k: (0, 0)),
            pl.BlockSpec((1, n), lambda k: (0, 0)),
        ],
        out_specs=pl.BlockSpec((bm, n), lambda k: (k, 0)),
    )(x, stats, gamma2, beta2)


# device time: 20505 ns/iter; 2.7600x vs baseline; 2.0270x over previous
import jax
import jax.numpy as jnp
from jax import lax
from jax.experimental import pallas as pl
from jax.experimental.pallas import tpu as pltpu

N_GLOBAL = 2048
EPS = 1e-5
K = 8


def kernel(x, gamma, beta):
    m, n = x.shape
    bm = m // K
    gamma2 = gamma.reshape(1, n)
    beta2 = beta.reshape(1, n)

    def stats_body(x_ref, stats_ref, part, send_buf, recv, send_sem, recv_sem):
        k = pl.program_id(0)
        xv = x_ref[...]
        s1 = jnp.sum(xv, axis=1, keepdims=True)
        s2 = jnp.sum(xv * xv, axis=1, keepdims=True)
        part[pl.ds(k * bm, bm), 0:1] = s1
        part[pl.ds(k * bm, bm), 1:2] = s2

        @pl.when(k == K - 1)
        def _():
            my_x = lax.axis_index("x")
            my_y = lax.axis_index("y")
            peer = (my_x, 1 - my_y)
            barrier = pltpu.get_barrier_semaphore()
            pl.semaphore_signal(
                barrier, inc=1, device_id=peer,
                device_id_type=pl.DeviceIdType.MESH,
            )
            pl.semaphore_wait(barrier, 1)

            send_buf[...] = jnp.transpose(part[...], (1, 0))
            rdma = pltpu.make_async_remote_copy(
                src_ref=send_buf,
                dst_ref=recv,
                send_sem=send_sem,
                recv_sem=recv_sem,
                device_id=peer,
                device_id_type=pl.DeviceIdType.MESH,
            )
            rdma.start()
            rdma.wait()

            rt = jnp.transpose(recv[...], (1, 0))
            tot1 = part[:, 0:1] + rt[:, 0:1]
            tot2 = part[:, 1:2] + rt[:, 1:2]
            mean = tot1 / N_GLOBAL
            var = tot2 / N_GLOBAL - mean * mean
            stats_ref[:, 0:1] = mean
            stats_ref[:, 1:2] = lax.rsqrt(var + EPS)

    stats = pl.pallas_call(
        stats_body,
        grid=(K,),
        out_shape=jax.ShapeDtypeStruct((m, 2), jnp.float32),
        in_specs=[pl.BlockSpec((bm, n), lambda k: (k, 0))],
        out_specs=pl.BlockSpec((m, 2), lambda k: (0, 0)),
        scratch_shapes=[
            pltpu.VMEM((m, 2), jnp.float32),
            pltpu.VMEM((2, m), jnp.float32),
            pltpu.VMEM((2, m), jnp.float32),
            pltpu.SemaphoreType.DMA,
            pltpu.SemaphoreType.DMA,
        ],
        compiler_params=pltpu.CompilerParams(collective_id=0),
    )(x)

    def norm_body(x_ref, stats_ref, g_ref, b_ref, o_ref):
        k = pl.program_id(0)
        mean = stats_ref[pl.ds(k * bm, bm), 0:1]
        rstd = stats_ref[pl.ds(k * bm, bm), 1:2]
        o_ref[...] = (
            (x_ref[...] - mean) * rstd * g_ref[...] + b_ref[...]
        ).astype(jnp.bfloat16)

    return pl.pallas_call(
        norm_body,
        grid=(K,),
        out_shape=jax.ShapeDtypeStruct((m, n), jnp.bfloat16),
        in_specs=[
            pl.BlockSpec((bm, n), lambda k: (k, 0)),
            pl.BlockSpec((m, 2), lambda k: (0, 0)),
            pl.BlockSpec((1, n), lambda k: (0, 0)),
            pl.BlockSpec((1, n), lambda k: (0, 0)),
        ],
        out_specs=pl.BlockSpec((bm, n), lambda k: (k, 0)),
    )(x, stats, gamma2, beta2)
